# baseline (speedup 1.0000x reference)
import jax
import jax.numpy as jnp
from jax.experimental import pallas as pl
from jax.experimental.pallas import tpu as pltpu

_SLOTS = 2
_LOOK = 1
_HSLOTS = 3
_HLOOK = 2
_PSPLITS = ((0, 256), (256, 256), (512, 256), (768, 257))


def _make_body(nb, nt, ph):
    n = nb * nt

    def _body(scal_ref, gate_ref, hs_ref, emb_ref, tile_ref, out_ref,
              hbuf_ref, hsem, obuf_ref, osem, ebuf_ref, esem, tbuf_ref, tsem):
        p, h = ebuf_ref.shape[-2:]
        it = pl.program_id(0)
        ik = pl.program_id(1)
        g = it * nb + ik

        base = 2 * nb
        fc0 = scal_ref[base + g]
        fc1 = scal_ref[base + g + 1]
        fcl0 = scal_ref[base + g + _LOOK]
        fcl1 = scal_ref[base + g + _LOOK + 1]
        slot_g = jax.lax.rem(fc1 - 1, _SLOTS)
        slot_n = jax.lax.rem(fcl1 - 1, _SLOTS)

        def _tile_copy(step, slot):
            k = jax.lax.rem(step, nb)
            tt = jax.lax.div(step, nb)
            row = scal_ref[k]
            return pltpu.make_async_copy(
                tile_ref.at[row, pl.ds(tt * ph, ph)],
                tbuf_ref.at[slot, 0, :],
                tsem.at[slot],
            )

        def _hs_copies(step, slot):
            k = jax.lax.rem(step, nb)
            tt = jax.lax.div(step, nb)
            bb = scal_ref[nb + k]
            return [
                pltpu.make_async_copy(
                    hs_ref.at[bb, tt, pl.ds(p0, pc), :],
                    hbuf_ref.at[slot, pl.ds(p0, pc), :],
                    hsem.at[slot, j],
                )
                for j, (p0, pc) in enumerate(_PSPLITS)
            ]

        def _out_copies(step, slot):
            k = jax.lax.rem(step, nb)
            tt = jax.lax.div(step, nb)
            bb = scal_ref[nb + k]
            return [
                pltpu.make_async_copy(
                    obuf_ref.at[slot, pl.ds(p0, pc), :],
                    out_ref.at[bb, tt, pl.ds(p0, pc), :],
                    osem.at[slot, j],
                )
                for j, (p0, pc) in enumerate(_PSPLITS)
            ]

        @pl.when(g == 0)
        def _():
            pltpu.make_async_copy(emb_ref, ebuf_ref, esem).start()
            for q in range(min(_HLOOK, n)):
                for c in _hs_copies(q, q % _HSLOTS):
                    c.start()
            for j in range(_LOOK):
                fa = scal_ref[base + j]
                fb = scal_ref[base + j + 1]

                @pl.when(fb != fa)
                def _():
                    _tile_copy(j, jax.lax.rem(fb - 1, _SLOTS)).start()

            pltpu.make_async_copy(emb_ref, ebuf_ref, esem).wait()

        @pl.when(g + _HLOOK < n)
        def _():
            for c in _hs_copies(g + _HLOOK, jax.lax.rem(g + _HLOOK, _HSLOTS)):
                c.start()

        @pl.when(fcl1 != fcl0)
        def _():
            _tile_copy(g + _LOOK, slot_n).start()

        for c in _hs_copies(g, jax.lax.rem(g, _HSLOTS)):
            c.wait()

        @pl.when(fc1 != fc0)
        def _():
            _tile_copy(g, slot_g).wait()

        oslot = jax.lax.rem(g, 2)

        @pl.when(g >= 2)
        def _():
            for c in _out_copies(g - 2, oslot):
                c.wait()

        gate = jnp.tanh(gate_ref[0])
        tile = tbuf_ref[slot_g].reshape(p, h)
        obuf_ref[oslot] = hbuf_ref[jax.lax.rem(g, _HSLOTS)] + (
            (1.0 - gate) * ebuf_ref[...] + gate * tile
        )

        for c in _out_copies(g, oslot):
            c.start()

        @pl.when(g == n - 1)
        def _():
            if n > 1:
                for c in _out_copies(g - 1, jax.lax.rem(g - 1, 2)):
                    c.wait()

            for c in _out_copies(g, oslot):
                c.wait()

    return _body


def kernel(hidden_state, aspect_ratio_ids, gate, embedding, tile_table):
    b, t, p, h = hidden_state.shape
    ph = p * h
    n = t * b

    ids = aspect_ratio_ids.astype(jnp.int32)
    perm = jnp.argsort(ids).astype(jnp.int32)
    sids = jnp.take(ids, perm)
    k_of_g = jnp.arange(n, dtype=jnp.int32) % b
    row_of_g = sids[k_of_g]
    prev_row = jnp.roll(row_of_g, 1)
    t_of_g = jnp.arange(n, dtype=jnp.int32) // b
    prev_t = jnp.roll(t_of_g, 1)
    nf = jnp.where(
        (jnp.arange(n) == 0) | (row_of_g != prev_row) | (t_of_g != prev_t), 1, 0
    ).astype(jnp.int32)
    cs = jnp.cumsum(nf).astype(jnp.int32)
    fcz = jnp.concatenate([
        jnp.zeros((1,), jnp.int32),
        cs,
        jnp.broadcast_to(cs[-1:], (_LOOK + 1,)),
    ])
    scal = jnp.concatenate([sids, perm, fcz])

    grid_spec = pltpu.PrefetchScalarGridSpec(
        num_scalar_prefetch=1,
        grid=(t, b),
        in_specs=[
            pl.BlockSpec(memory_space=pltpu.SMEM),
            pl.BlockSpec(memory_space=pl.ANY),
            pl.BlockSpec(memory_space=pl.ANY),
            pl.BlockSpec(memory_space=pl.ANY),
        ],
        out_specs=pl.BlockSpec(memory_space=pl.ANY),
        scratch_shapes=[
            pltpu.VMEM((_HSLOTS, p, h), jnp.float32),
            pltpu.SemaphoreType.DMA((_HSLOTS, len(_PSPLITS))),
            pltpu.VMEM((2, p, h), jnp.float32),
            pltpu.SemaphoreType.DMA((2, len(_PSPLITS))),
            pltpu.VMEM((p, h), jnp.float32),
            pltpu.SemaphoreType.DMA,
            pltpu.VMEM((_SLOTS, 1, ph), jnp.float32),
            pltpu.SemaphoreType.DMA((_SLOTS,)),
        ],
    )

    return pl.pallas_call(
        _make_body(b, t, ph),
        grid_spec=grid_spec,
        out_shape=jax.ShapeDtypeStruct(hidden_state.shape, hidden_state.dtype),
        compiler_params=pltpu.CompilerParams(
            dimension_semantics=("arbitrary", "arbitrary"),
        ),
    )(scal, gate, hidden_state, embedding, tile_table)


# speedup vs baseline: 1.0116x; 1.0116x over previous
import jax
import jax.numpy as jnp
from jax.experimental import pallas as pl
from jax.experimental.pallas import tpu as pltpu

_SLOTS = 2
_LOOK = 1


def _make_body(nb, nt, ph):
    def _body(scal_ref, gate_ref, hs_ref, emb_ref, tile_ref, out_ref,
              tbuf_ref, sems, ebuf_ref, esem):
        p, h = ebuf_ref.shape
        it = pl.program_id(0)
        ik = pl.program_id(1)
        g = it * nb + ik

        base = 2 * nb
        fc0 = scal_ref[base + g]
        fc1 = scal_ref[base + g + 1]
        fcl0 = scal_ref[base + g + _LOOK]
        fcl1 = scal_ref[base + g + _LOOK + 1]
        slot_g = jax.lax.rem(fc1 - 1, _SLOTS)
        slot_n = jax.lax.rem(fcl1 - 1, _SLOTS)

        def _copy(step, slot):
            k = jax.lax.rem(step, nb)
            tt = jax.lax.div(step, nb)
            row = scal_ref[k]
            return pltpu.make_async_copy(
                tile_ref.at[row, pl.ds(tt * ph, ph)],
                tbuf_ref.at[slot, 0, :],
                sems.at[slot],
            )

        @pl.when(g == 0)
        def _():
            pltpu.make_async_copy(emb_ref, ebuf_ref, esem).start()
            for j in range(_LOOK):
                fa = scal_ref[base + j]
                fb = scal_ref[base + j + 1]

                @pl.when(fb != fa)
                def _():
                    _copy(j, jax.lax.rem(fb - 1, _SLOTS)).start()

            pltpu.make_async_copy(emb_ref, ebuf_ref, esem).wait()

        @pl.when(fcl1 != fcl0)
        def _():
            _copy(g + _LOOK, slot_n).start()

        @pl.when(fc1 != fc0)
        def _():
            _copy(g, slot_g).wait()

        gate = jnp.tanh(gate_ref[0])
        tile = tbuf_ref[slot_g].reshape(p, h)
        out_ref[...] = hs_ref[...] + (
            (1.0 - gate) * ebuf_ref[...] + gate * tile
        )[None, None]

    return _body


def kernel(hidden_state, aspect_ratio_ids, gate, embedding, tile_table):
    b, t, p, h = hidden_state.shape
    ph = p * h
    n = t * b

    ids = aspect_ratio_ids.astype(jnp.int32)
    perm = jnp.argsort(ids).astype(jnp.int32)
    sids = jnp.take(ids, perm)
    k_of_g = jnp.arange(n, dtype=jnp.int32) % b
    row_of_g = sids[k_of_g]
    prev_row = jnp.roll(row_of_g, 1)
    t_of_g = jnp.arange(n, dtype=jnp.int32) // b
    prev_t = jnp.roll(t_of_g, 1)
    nf = jnp.where(
        (jnp.arange(n) == 0) | (row_of_g != prev_row) | (t_of_g != prev_t), 1, 0
    ).astype(jnp.int32)
    cs = jnp.cumsum(nf).astype(jnp.int32)
    fcz = jnp.concatenate([
        jnp.zeros((1,), jnp.int32),
        cs,
        jnp.broadcast_to(cs[-1:], (_LOOK + 1,)),
    ])
    scal = jnp.concatenate([sids, perm, fcz])

    grid_spec = pltpu.PrefetchScalarGridSpec(
        num_scalar_prefetch=1,
        grid=(t, b),
        in_specs=[
            pl.BlockSpec(memory_space=pltpu.SMEM),
            pl.BlockSpec((1, 1, p, h), lambda it, ik, s: (s[b + ik], it, 0, 0)),
            pl.BlockSpec(memory_space=pl.ANY),
            pl.BlockSpec(memory_space=pl.ANY),
        ],
        out_specs=pl.BlockSpec((1, 1, p, h), lambda it, ik, s: (s[b + ik], it, 0, 0)),
        scratch_shapes=[
            pltpu.VMEM((_SLOTS, 1, ph), jnp.float32),
            pltpu.SemaphoreType.DMA((_SLOTS,)),
            pltpu.VMEM((p, h), jnp.float32),
            pltpu.SemaphoreType.DMA,
        ],
    )

    return pl.pallas_call(
        _make_body(b, t, ph),
        grid_spec=grid_spec,
        out_shape=jax.ShapeDtypeStruct(hidden_state.shape, hidden_state.dtype),
        compiler_params=pltpu.CompilerParams(
            dimension_semantics=("arbitrary", "arbitrary"),
        ),
    )(scal, gate, hidden_state, embedding, tile_table)
